# Initial kernel scaffold; baseline (speedup 1.0000x reference)
import jax
import jax.numpy as jnp
from jax.experimental import pallas as pl


def kernel(station_ids, geometry_ids, station_table, geometry_table, gamma, beta, W1, b1, W2, b2):
    raise NotImplementedError("write your pallas kernel here")


# speedup vs baseline: 1.6797x; 1.6797x over previous
import functools
import math

import jax
import jax.numpy as jnp
from jax import lax
from jax.experimental import pallas as pl
from jax.experimental.pallas import tpu as pltpu
from jax.experimental.pallas import tpu_sc as plsc

DIM = 64
HID = 128
G = 128
TC_ROWS = 2048


def _sc_gather(ids_s, ids_g, station_table, geometry_table, nw, steps):
    n_rows = nw * steps * G
    mesh = plsc.VectorSubcoreMesh(core_axis_name="c", subcore_axis_name="s")
    nc = mesh.num_cores

    def body(sid_hbm, gid_hbm, stab_hbm, gtab_hbm, outs_hbm, outg_hbm,
             sidx, gidx, bufs, bufg, sem):
        wid = lax.axis_index("s") * nc + lax.axis_index("c")
        pltpu.sync_copy(sid_hbm.at[wid], sidx)
        pltpu.sync_copy(gid_hbm.at[wid], gidx)
        row0 = wid * (steps * G)

        def step(j, carry):
            a = pltpu.async_copy(stab_hbm.at[sidx.at[j]], bufs, sem)
            b = pltpu.async_copy(gtab_hbm.at[gidx.at[j]], bufg, sem)
            a.wait()
            b.wait()
            base = row0 + j * G
            pltpu.sync_copy(bufs, outs_hbm.at[pl.ds(base, G)])
            pltpu.sync_copy(bufg, outg_hbm.at[pl.ds(base, G)])
            return carry

        lax.fori_loop(0, steps, step, 0)

    f = pl.kernel(
        body,
        out_type=(
            jax.ShapeDtypeStruct((n_rows, DIM), jnp.float32),
            jax.ShapeDtypeStruct((n_rows, DIM), jnp.float32),
        ),
        mesh=mesh,
        scratch_types=[
            pltpu.VMEM((steps, G), jnp.int32),
            pltpu.VMEM((steps, G), jnp.int32),
            pltpu.VMEM((G, DIM), jnp.float32),
            pltpu.VMEM((G, DIM), jnp.float32),
            pltpu.SemaphoreType.DMA,
        ],
        compiler_params=pltpu.CompilerParams(use_tc_tiling_on_sc=False),
    )
    return f(ids_s, ids_g, station_table, geometry_table)


def _mlp_body(es_ref, eg_ref, gamma_ref, beta_ref, w1_ref, b1_ref, w2_ref,
              b2_ref, o_ref):
    x = es_ref[...] + eg_ref[...]
    mu = jnp.mean(x, axis=-1, keepdims=True)
    xc = x - mu
    var = jnp.mean(xc * xc, axis=-1, keepdims=True)
    h = xc * lax.rsqrt(var + 1e-5) * gamma_ref[...] + beta_ref[...]
    h = jnp.dot(h, w1_ref[...], preferred_element_type=jnp.float32) + b1_ref[...]
    h = 0.5 * h * (1.0 + lax.erf(h * (1.0 / math.sqrt(2.0))))
    o_ref[...] = jnp.dot(h, w2_ref[...], preferred_element_type=jnp.float32) + b2_ref[...]


def _tc_mlp(es, eg, gamma, beta, W1, b1, W2, b2):
    n_rows = es.shape[0]
    grid = (n_rows // TC_ROWS,)
    full = lambda shape: pl.BlockSpec(shape, lambda i: (0,) * len(shape))
    return pl.pallas_call(
        _mlp_body,
        grid=grid,
        in_specs=[
            pl.BlockSpec((TC_ROWS, DIM), lambda i: (i, 0)),
            pl.BlockSpec((TC_ROWS, DIM), lambda i: (i, 0)),
            full((1, DIM)),
            full((1, DIM)),
            full((DIM, HID)),
            full((1, HID)),
            full((HID, DIM)),
            full((1, DIM)),
        ],
        out_specs=pl.BlockSpec((TC_ROWS, DIM), lambda i: (i, 0)),
        out_shape=jax.ShapeDtypeStruct((n_rows, DIM), jnp.float32),
        compiler_params=pltpu.CompilerParams(
            dimension_semantics=("parallel",),
        ),
    )(es, eg, gamma.reshape(1, DIM), beta.reshape(1, DIM), W1,
      b1.reshape(1, HID), W2, b2.reshape(1, DIM))


def kernel(station_ids, geometry_ids, station_table, geometry_table, gamma,
           beta, W1, b1, W2, b2):
    B, L = station_ids.shape
    n = B * L
    nw = 32
    steps = n // (nw * G)
    assert steps * nw * G == n

    ids_s = station_ids.reshape(nw, steps, G).astype(jnp.int32)
    ids_g = geometry_ids.reshape(nw, steps, G).astype(jnp.int32)
    es, eg = _sc_gather(ids_s, ids_g, station_table, geometry_table, nw, steps)
    out = _tc_mlp(es, eg, gamma, beta, W1, b1, W2, b2)
    return out.reshape(B, L, DIM)


# speedup vs baseline: 1.7807x; 1.0602x over previous
import functools
import math

import jax
import jax.numpy as jnp
from jax import lax
from jax.experimental import pallas as pl
from jax.experimental.pallas import tpu as pltpu
from jax.experimental.pallas import tpu_sc as plsc

DIM = 64
HID = 128
G = 128
TC_ROWS = 1024


def _sc_gather(ids_s, ids_g, station_table, geometry_table, nw, steps):
    n_rows = nw * steps * G
    mesh = plsc.VectorSubcoreMesh(core_axis_name="c", subcore_axis_name="s")
    nc = mesh.num_cores

    def body(sid_hbm, gid_hbm, stab_hbm, gtab_hbm, outs_hbm, outg_hbm,
             sidx, gidx, bufs, bufg, sem):
        wid = lax.axis_index("s") * nc + lax.axis_index("c")
        pltpu.sync_copy(sid_hbm.at[wid], sidx)
        pltpu.sync_copy(gid_hbm.at[wid], gidx)
        row0 = wid * (steps * G)

        def step(j, carry):
            a = pltpu.async_copy(stab_hbm.at[sidx.at[j]], bufs, sem)
            b = pltpu.async_copy(gtab_hbm.at[gidx.at[j]], bufg, sem)
            a.wait()
            b.wait()
            base = row0 + j * G
            pltpu.sync_copy(bufs, outs_hbm.at[pl.ds(base, G)])
            pltpu.sync_copy(bufg, outg_hbm.at[pl.ds(base, G)])
            return carry

        lax.fori_loop(0, steps, step, 0)

    f = pl.kernel(
        body,
        out_type=(
            jax.ShapeDtypeStruct((n_rows, DIM), jnp.float32),
            jax.ShapeDtypeStruct((n_rows, DIM), jnp.float32),
        ),
        mesh=mesh,
        scratch_types=[
            pltpu.VMEM((steps, G), jnp.int32),
            pltpu.VMEM((steps, G), jnp.int32),
            pltpu.VMEM((G, DIM), jnp.float32),
            pltpu.VMEM((G, DIM), jnp.float32),
            pltpu.SemaphoreType.DMA,
        ],
        compiler_params=pltpu.CompilerParams(use_tc_tiling_on_sc=False),
    )
    return f(ids_s, ids_g, station_table, geometry_table)


def _ln_mlp_half(x, w1g, b1b, w2, b2):
    mu = jnp.mean(x, axis=-1, keepdims=True)
    xc = x - mu
    var = jnp.mean(xc * xc, axis=-1, keepdims=True)
    y = xc * lax.rsqrt(var + 1e-5)
    h = jnp.dot(y, w1g, preferred_element_type=jnp.float32) + b1b
    h = 0.5 * h * (1.0 + lax.erf(h * (1.0 / math.sqrt(2.0))))
    return jnp.dot(h, w2, preferred_element_type=jnp.float32) + b2


def _mlp_body(es_ref, eg_ref, w1g_ref, b1b_ref, w2_ref, b2_ref, o_ref):
    x = es_ref[...] + eg_ref[...]
    z0 = _ln_mlp_half(x[:, :DIM], w1g_ref[...], b1b_ref[...], w2_ref[...],
                      b2_ref[...])
    z1 = _ln_mlp_half(x[:, DIM:], w1g_ref[...], b1b_ref[...], w2_ref[...],
                      b2_ref[...])
    o_ref[...] = jnp.concatenate([z0, z1], axis=1)


def _tc_mlp(es, eg, gamma, beta, W1, b1, W2, b2):
    n_wide = es.shape[0] // 2
    esw = es.reshape(n_wide, 2 * DIM)
    egw = eg.reshape(n_wide, 2 * DIM)
    w1g = gamma[:, None] * W1
    b1b = (beta @ W1 + b1).reshape(1, HID)
    grid = (n_wide // TC_ROWS,)
    full = lambda shape: pl.BlockSpec(shape, lambda i: (0,) * len(shape))
    out = pl.pallas_call(
        _mlp_body,
        grid=grid,
        in_specs=[
            pl.BlockSpec((TC_ROWS, 2 * DIM), lambda i: (i, 0)),
            pl.BlockSpec((TC_ROWS, 2 * DIM), lambda i: (i, 0)),
            full((DIM, HID)),
            full((1, HID)),
            full((HID, DIM)),
            full((1, DIM)),
        ],
        out_specs=pl.BlockSpec((TC_ROWS, 2 * DIM), lambda i: (i, 0)),
        out_shape=jax.ShapeDtypeStruct((n_wide, 2 * DIM), jnp.float32),
        compiler_params=pltpu.CompilerParams(
            dimension_semantics=("parallel",),
        ),
    )(esw, egw, w1g, b1b, W2, b2.reshape(1, DIM))
    return out.reshape(n_wide * 2, DIM)


def kernel(station_ids, geometry_ids, station_table, geometry_table, gamma,
           beta, W1, b1, W2, b2):
    B, L = station_ids.shape
    n = B * L
    nw = 32
    steps = n // (nw * G)
    assert steps * nw * G == n

    ids_s = station_ids.reshape(nw, steps, G).astype(jnp.int32)
    ids_g = geometry_ids.reshape(nw, steps, G).astype(jnp.int32)
    es, eg = _sc_gather(ids_s, ids_g, station_table, geometry_table, nw, steps)
    out = _tc_mlp(es, eg, gamma, beta, W1, b1, W2, b2)
    return out.reshape(B, L, DIM)


# speedup vs baseline: 2.1834x; 1.2261x over previous
import functools
import math

import jax
import jax.numpy as jnp
from jax import lax
from jax.experimental import pallas as pl
from jax.experimental.pallas import tpu as pltpu
from jax.experimental.pallas import tpu_sc as plsc

DIM = 64
HID = 128
G = 128


def _sc_gather(ids_s, ids_g, station_table, geometry_table, nw, steps):
    n_rows = nw * steps * G
    mesh = plsc.VectorSubcoreMesh(core_axis_name="c", subcore_axis_name="s")
    nc = mesh.num_cores

    def body(sid_hbm, gid_hbm, stab_hbm, gtab_hbm, outs_hbm, outg_hbm,
             sidx, gidx, bufs, bufg, gsem, wsem):
        wid = lax.axis_index("s") * nc + lax.axis_index("c")
        pltpu.sync_copy(sid_hbm.at[wid], sidx)
        pltpu.sync_copy(gid_hbm.at[wid], gidx)
        row0 = wid * (steps * G)

        pltpu.async_copy(stab_hbm.at[sidx.at[0]], bufs.at[0], gsem)
        pltpu.async_copy(gtab_hbm.at[gidx.at[0]], bufg.at[0], gsem)

        def step(j, carry):
            slot = lax.rem(j, 2)
            nxt = lax.rem(j + 1, 2)
            pltpu.make_async_copy(stab_hbm.at[sidx.at[j]], bufs.at[slot],
                                  gsem).wait()
            pltpu.make_async_copy(gtab_hbm.at[gidx.at[j]], bufg.at[slot],
                                  gsem).wait()

            @pl.when(j + 1 < steps)
            def _():
                pltpu.async_copy(stab_hbm.at[sidx.at[j + 1]], bufs.at[nxt],
                                 gsem)
                pltpu.async_copy(gtab_hbm.at[gidx.at[j + 1]], bufg.at[nxt],
                                 gsem)

            base = row0 + j * G
            pltpu.sync_copy(bufs.at[slot], outs_hbm.at[pl.ds(base, G)])
            pltpu.sync_copy(bufg.at[slot], outg_hbm.at[pl.ds(base, G)])
            return carry

        lax.fori_loop(0, steps, step, 0)

    f = pl.kernel(
        body,
        out_type=(
            jax.ShapeDtypeStruct((n_rows, DIM), jnp.float32),
            jax.ShapeDtypeStruct((n_rows, DIM), jnp.float32),
        ),
        mesh=mesh,
        scratch_types=[
            pltpu.VMEM((steps, G), jnp.int32),
            pltpu.VMEM((steps, G), jnp.int32),
            pltpu.VMEM((2, G, DIM), jnp.float32),
            pltpu.VMEM((2, G, DIM), jnp.float32),
            pltpu.SemaphoreType.DMA,
            pltpu.SemaphoreType.DMA,
        ],
        compiler_params=pltpu.CompilerParams(use_tc_tiling_on_sc=False),
    )
    return f(ids_s, ids_g, station_table, geometry_table)


def _ln_mlp_half_t(x, w1g, b1bt, w2, b2t):
    mu = jnp.mean(x, axis=-1, keepdims=True)
    xc = x - mu
    var = jnp.mean(xc * xc, axis=-1, keepdims=True)
    y = xc * lax.rsqrt(var + 1e-5)
    ht = lax.dot_general(w1g, y, (((0,), (1,)), ((), ())),
                         preferred_element_type=jnp.float32) + b1bt
    ht = 0.5 * ht * (1.0 + lax.erf(ht * (1.0 / math.sqrt(2.0))))
    return lax.dot_general(w2, ht, (((0,), (0,)), ((), ())),
                           preferred_element_type=jnp.float32) + b2t


def _mlp_body(es_ref, eg_ref, w1g_ref, b1bt_ref, w2_ref, b2t_ref, o_ref):
    x = es_ref[...] + eg_ref[...]
    z0t = _ln_mlp_half_t(x[:, :DIM], w1g_ref[...], b1bt_ref[...], w2_ref[...],
                         b2t_ref[...])
    z1t = _ln_mlp_half_t(x[:, DIM:], w1g_ref[...], b1bt_ref[...], w2_ref[...],
                         b2t_ref[...])
    r = x.shape[0]
    o_ref[0, :, 0:r] = z0t
    o_ref[0, :, r:2 * r] = z1t


def _tc_mlp(es, eg, gamma, beta, W1, b1, W2, b2, B, L):
    n_wide = es.shape[0] // 2
    wide_per_l = B // 2
    esw = es.reshape(n_wide, 2 * DIM)
    egw = eg.reshape(n_wide, 2 * DIM)
    w1g = gamma[:, None] * W1
    b1bt = (beta @ W1 + b1).reshape(HID, 1)
    b2t = b2.reshape(DIM, 1)
    grid = (L,)
    full = lambda shape: pl.BlockSpec(shape, lambda i: (0,) * len(shape))
    out = pl.pallas_call(
        _mlp_body,
        grid=grid,
        in_specs=[
            pl.BlockSpec((wide_per_l, 2 * DIM), lambda i: (i, 0)),
            pl.BlockSpec((wide_per_l, 2 * DIM), lambda i: (i, 0)),
            full((DIM, HID)),
            full((HID, 1)),
            full((HID, DIM)),
            full((DIM, 1)),
        ],
        out_specs=pl.BlockSpec((1, DIM, B), lambda i: (i, 0, 0)),
        out_shape=jax.ShapeDtypeStruct((L, DIM, B), jnp.float32),
        compiler_params=pltpu.CompilerParams(
            dimension_semantics=("parallel",),
        ),
    )(esw, egw, w1g, b1bt, W2, b2t)
    return jnp.transpose(out, (2, 0, 1))


def _permute_ids(ids, nw, steps, B, L):
    t = ids.T.astype(jnp.int32)
    t = t.reshape(L, 2, B // 2)
    t = jnp.transpose(t, (0, 2, 1))
    return t.reshape(nw, steps, G)


def kernel(station_ids, geometry_ids, station_table, geometry_table, gamma,
           beta, W1, b1, W2, b2):
    B, L = station_ids.shape
    n = B * L
    nw = 32
    steps = n // (nw * G)
    assert steps * nw * G == n

    ids_s = _permute_ids(station_ids, nw, steps, B, L)
    ids_g = _permute_ids(geometry_ids, nw, steps, B, L)
    es, eg = _sc_gather(ids_s, ids_g, station_table, geometry_table, nw, steps)
    return _tc_mlp(es, eg, gamma, beta, W1, b1, W2, b2, B, L)
